# Initial kernel scaffold; baseline (speedup 1.0000x reference)
import functools

import jax
import jax.numpy as jnp
from jax import lax
from jax.experimental import pallas as pl
from jax.experimental.pallas import tpu as pltpu
from jax.experimental.pallas import tpu_sc as plsc

N_NODES = 10000
N_EDGES = 320000
D = 128

NC = 2
NS = 16
NW = NC * NS
CHUNK = 128
NCHUNKS = N_EDGES // CHUNK
ROWS_PER_SUB = N_NODES // NS
ZROWS = 125


def _matmul_body(x_ref, w_ref, o_ref):
    o_ref[...] = jnp.dot(x_ref[...], w_ref[...],
                         preferred_element_type=jnp.float32)


def _matmul(x, W):
    blk = 1000
    grid = N_NODES // blk
    return pl.pallas_call(
        _matmul_body,
        grid=(grid,),
        in_specs=[
            pl.BlockSpec((blk, D), lambda i: (i, 0)),
            pl.BlockSpec((D, D), lambda i: (0, 0)),
        ],
        out_specs=pl.BlockSpec((blk, D), lambda i: (i, 0)),
        out_shape=jax.ShapeDtypeStruct((N_NODES, D), jnp.float32),
    )(x, W)


def _combine_body(p_ref, o_ref):
    o_ref[...] = jnp.maximum(p_ref[0] + p_ref[1], 0.0)


def _combine(partials):
    blk = 1000
    grid = N_NODES // blk
    return pl.pallas_call(
        _combine_body,
        grid=(grid,),
        in_specs=[pl.BlockSpec((2, blk, D), lambda i: (0, i, 0))],
        out_specs=pl.BlockSpec((blk, D), lambda i: (i, 0)),
        out_shape=jax.ShapeDtypeStruct((N_NODES, D), jnp.float32),
    )(partials)


def _sc_body(support_hbm, src_hbm, dst_hbm, ew_hbm, out_hbm,
             acc_spmem, src_idx, dst_idx, wbuf, rows, zbuf, sem):
    core = lax.axis_index("c")
    sub = lax.axis_index("s")
    wid = sub * NC + core

    @pl.loop(0, ZROWS)
    def _(r):
        for g in range(D // 16):
            zbuf[r, pl.ds(g * 16, 16)] = jnp.zeros((16,), jnp.float32)

    base = sub * ROWS_PER_SUB
    for k in range(ROWS_PER_SUB // ZROWS):
        pltpu.sync_copy(zbuf, acc_spmem.at[pl.ds(base + k * ZROWS, ZROWS)])
    plsc.subcore_barrier()

    @pl.loop(wid, NCHUNKS, step=NW)
    def _(c):
        e0 = c * CHUNK
        pltpu.sync_copy(src_hbm.at[pl.ds(e0, CHUNK)], src_idx)
        pltpu.sync_copy(dst_hbm.at[pl.ds(e0, CHUNK)], dst_idx)
        pltpu.sync_copy(ew_hbm.at[pl.ds(e0, CHUNK)], wbuf)
        pltpu.async_copy(support_hbm.at[src_idx], rows, sem).wait()

        @pl.loop(0, CHUNK)
        def _(e):
            w = jnp.full((16,), wbuf[e], jnp.float32)
            for g in range(D // 16):
                sl = pl.ds(g * 16, 16)
                rows[e, sl] = rows[e, sl] * w

        pltpu.sync_copy(rows, acc_spmem.at[dst_idx], add=True)

    plsc.subcore_barrier()

    ob = core * N_NODES + base
    for k in range(ROWS_PER_SUB // ZROWS):
        pltpu.sync_copy(acc_spmem.at[pl.ds(base + k * ZROWS, ZROWS)],
                        out_hbm.at[pl.ds(ob + k * ZROWS, ZROWS)])


def _sc_spmm(support, src, dst, ew):
    mesh = plsc.VectorSubcoreMesh(core_axis_name="c", subcore_axis_name="s")
    f = pl.kernel(
        _sc_body,
        out_type=jax.ShapeDtypeStruct((NC * N_NODES, D), jnp.float32),
        mesh=mesh,
        scratch_types=[
            pltpu.VMEM_SHARED((N_NODES, D), jnp.float32),
            pltpu.VMEM((CHUNK,), jnp.int32),
            pltpu.VMEM((CHUNK,), jnp.int32),
            pltpu.VMEM((CHUNK,), jnp.float32),
            pltpu.VMEM((CHUNK, D), jnp.float32),
            pltpu.VMEM((ZROWS, D), jnp.float32),
            pltpu.SemaphoreType.DMA,
        ],
    )
    return f(support, src, dst, ew)


def kernel(x, edge_index, edge_weight, W):
    support = _matmul(x, W)
    dst = edge_index[0]
    src = edge_index[1]
    partials = _sc_spmm(support, src, dst, edge_weight)
    return _combine(partials.reshape(NC, N_NODES, D))


# speedup vs baseline: 5.3932x; 5.3932x over previous
import functools

import jax
import jax.numpy as jnp
from jax import lax
from jax.experimental import pallas as pl
from jax.experimental.pallas import tpu as pltpu
from jax.experimental.pallas import tpu_sc as plsc

N_NODES = 10000
N_EDGES = 320000
D = 128

NC = 2
NS = 16
NW = NC * NS
CHUNK = 128
NCHUNKS = N_EDGES // CHUNK
ROWS_PER_SUB = 624
TAIL_ROWS = N_NODES - NS * ROWS_PER_SUB
ZROWS = 208


def _matmul_body(x_ref, w_ref, o_ref):
    o_ref[...] = jnp.dot(x_ref[...], w_ref[...],
                         preferred_element_type=jnp.float32)


def _matmul(x, W):
    blk = 1000
    grid = N_NODES // blk
    return pl.pallas_call(
        _matmul_body,
        grid=(grid,),
        in_specs=[
            pl.BlockSpec((blk, D), lambda i: (i, 0)),
            pl.BlockSpec((D, D), lambda i: (0, 0)),
        ],
        out_specs=pl.BlockSpec((blk, D), lambda i: (i, 0)),
        out_shape=jax.ShapeDtypeStruct((N_NODES, D), jnp.float32),
    )(x, W)


def _combine_body(p_ref, o_ref):
    o_ref[...] = jnp.maximum(p_ref[0] + p_ref[1], 0.0)


def _combine(partials):
    blk = 1000
    grid = N_NODES // blk
    return pl.pallas_call(
        _combine_body,
        grid=(grid,),
        in_specs=[pl.BlockSpec((2, blk, D), lambda i: (0, i, 0))],
        out_specs=pl.BlockSpec((blk, D), lambda i: (i, 0)),
        out_shape=jax.ShapeDtypeStruct((N_NODES, D), jnp.float32),
    )(partials)


def _sc_body(support_hbm, src_hbm, dst_hbm, ew_hbm, out_hbm,
             acc_spmem, src_idx, dst_idx, wbuf, rows, zbuf, sem):
    core = lax.axis_index("c")
    sub = lax.axis_index("s")
    wid = sub * NC + core

    @pl.loop(0, ZROWS)
    def _(r):
        for g in range(D // 16):
            zbuf[r, pl.ds(g * 16, 16)] = jnp.zeros((16,), jnp.float32)

    base = sub * ROWS_PER_SUB
    for k in range(ROWS_PER_SUB // ZROWS):
        pltpu.sync_copy(zbuf, acc_spmem.at[pl.ds(base + k * ZROWS, ZROWS)])

    @pl.when(sub == NS - 1)
    def _():
        pltpu.sync_copy(zbuf.at[pl.ds(0, TAIL_ROWS)],
                        acc_spmem.at[pl.ds(NS * ROWS_PER_SUB, TAIL_ROWS)])

    plsc.subcore_barrier()

    @pl.loop(wid, NCHUNKS, step=NW)
    def _(c):
        e0 = c * CHUNK
        pltpu.sync_copy(src_hbm.at[pl.ds(e0, CHUNK)], src_idx)
        pltpu.sync_copy(dst_hbm.at[pl.ds(e0, CHUNK)], dst_idx)
        pltpu.sync_copy(ew_hbm.at[pl.ds(e0, CHUNK)], wbuf)
        pltpu.async_copy(support_hbm.at[src_idx], rows, sem).wait()

        @pl.loop(0, CHUNK, step=16)
        def _(eg):
            w16 = wbuf[pl.ds(eg, 16)]
            for j in range(16):
                bw = jnp.full((16,), w16[j], jnp.float32)
                for g in range(D // 16):
                    sl = pl.ds(g * 16, 16)
                    rows[eg + j, sl] = rows[eg + j, sl] * bw

        pltpu.sync_copy(rows, acc_spmem.at[dst_idx], add=True)

    plsc.subcore_barrier()

    ob = core * N_NODES + base
    for k in range(ROWS_PER_SUB // ZROWS):
        pltpu.sync_copy(acc_spmem.at[pl.ds(base + k * ZROWS, ZROWS)],
                        out_hbm.at[pl.ds(ob + k * ZROWS, ZROWS)])

    @pl.when(sub == NS - 1)
    def _():
        pltpu.sync_copy(acc_spmem.at[pl.ds(NS * ROWS_PER_SUB, TAIL_ROWS)],
                        out_hbm.at[pl.ds(core * N_NODES + NS * ROWS_PER_SUB,
                                         TAIL_ROWS)])


def _sc_spmm(support, src, dst, ew):
    mesh = plsc.VectorSubcoreMesh(core_axis_name="c", subcore_axis_name="s")
    f = pl.kernel(
        _sc_body,
        out_type=jax.ShapeDtypeStruct((NC * N_NODES, D), jnp.float32),
        mesh=mesh,
        scratch_types=[
            pltpu.VMEM_SHARED((N_NODES, D), jnp.float32),
            pltpu.VMEM((CHUNK,), jnp.int32),
            pltpu.VMEM((CHUNK,), jnp.int32),
            pltpu.VMEM((CHUNK,), jnp.float32),
            pltpu.VMEM((CHUNK, D), jnp.float32),
            pltpu.VMEM((ZROWS, D), jnp.float32),
            pltpu.SemaphoreType.DMA,
        ],
    )
    return f(support, src, dst, ew)


def kernel(x, edge_index, edge_weight, W):
    support = _matmul(x, W)
    dst = edge_index[0]
    src = edge_index[1]
    partials = _sc_spmm(support, src, dst, edge_weight)
    return _combine(partials.reshape(NC, N_NODES, D))
